# baseline (speedup 1.0000x reference)
import jax
import jax.numpy as jnp
from jax import lax
from jax.experimental import pallas as pl
from jax.experimental.pallas import tpu as pltpu
from jax.experimental.pallas import tpu_sc as plsc

_NC, _NS, _L = 2, 16, 16

_LEVELS = (
    (256, 128, 128, 4.0, 0, 2),
    (512, 64, 64, 8.0, 2, 4),
    (1024, 32, 32, 16.0, 6, 8),
    (2048, 16, 16, 32.0, 14, 16),
)
_DEC = (64, 256, 256, 2.0, 30, 2)


def _axis(lo, hi, t, n):
    if n > 1:
        return lo + (hi - lo) * (float(t) / float(n - 1))
    return (lo + hi) * 0.5


def _split(pos, limit):
    pos = jnp.clip(pos, 0.0, float(limit - 1))
    i0 = pos.astype(jnp.int32)
    frac = pos - i0.astype(jnp.float32)
    i1 = jnp.minimum(i0 + 1, limit - 1)
    return i0, i1, frac


def _body(f1, f2, f3, f4, fd, bbh, o1, o2, o3, o4, od,
          bbv, ia, ib, ic, id_, va, vb, vc, vd, outv, sem):
    wid = lax.axis_index("c") * _NS + lax.axis_index("s")
    pltpu.sync_copy(bbh, bbv)
    bb = (bbv[0], bbv[1], bbv[2], bbv[3])
    iota = lax.iota(jnp.int32, _L)
    feats = (f1, f2, f3, f4)
    outs = (o1, o2, o3, o4)

    def gather4(feat):
        cps = [pltpu.async_copy(feat.at[idx], dst, sem)
               for idx, dst in ((ia, va), (ib, vb), (ic, vc), (id_, vd))]
        for cp in cps:
            cp.wait()

    for lvl, (C, H, W, s, t_lo, n_t) in enumerate(_LEVELS):
        @pl.when((wid >= t_lo) & (wid < t_lo + n_t))
        def _(lvl=lvl, H=H, W=W, s=s, t_lo=t_lo):
            inv = 1.0 / s
            y0, y1i, ly = _split(_axis(bb[1] * inv, bb[3] * inv, 0, 1), H)
            x0, x1i, lx = _split(_axis(bb[0] * inv, bb[2] * inv, 0, 1), W)
            ch_base = pl.multiple_of((wid - t_lo) * 128, 128)
            chv = jnp.full((_L,), ch_base, jnp.int32)
            for k in range(8):
                plane = (chv + (k * _L) + iota) * (H * W)
                sl = pl.ds(k * _L, _L)
                ia[sl] = plane + y0 * W + x0
                ib[sl] = plane + y0 * W + x1i
                ic[sl] = plane + y1i * W + x0
                id_[sl] = plane + y1i * W + x1i
            gather4(feats[lvl])
            for k in range(8):
                sl = pl.ds(k * _L, _L)
                top = va[sl] * (1.0 - lx) + vb[sl] * lx
                bot = vc[sl] * (1.0 - lx) + vd[sl] * lx
                outv[sl] = top * (1.0 - ly) + bot * ly
            pltpu.sync_copy(outv, outs[lvl].at[pl.ds(ch_base, 128)])

    C, H, W, s, t_lo, n_t = _DEC

    @pl.when(wid >= t_lo)
    def _():
        inv = 1.0 / s
        ys = [_split(_axis(bb[1] * inv, bb[3] * inv, i, 2), H) for i in range(2)]
        xs = [_split(_axis(bb[0] * inv, bb[2] * inv, j, 2), W) for j in range(2)]
        ch_base = pl.multiple_of((wid - t_lo) * 32, 32)
        chv = jnp.full((_L,), ch_base, jnp.int32)
        lws = []
        for k in range(8):
            ev = (k * _L) + iota
            iv = (ev >> 1) & 1
            jv = ev & 1
            y0 = jnp.where(iv == 1, ys[1][0], ys[0][0])
            y1i = jnp.where(iv == 1, ys[1][1], ys[0][1])
            x0 = jnp.where(jv == 1, xs[1][0], xs[0][0])
            x1i = jnp.where(jv == 1, xs[1][1], xs[0][1])
            lws.append((jnp.where(iv == 1, ys[1][2], ys[0][2]),
                        jnp.where(jv == 1, xs[1][2], xs[0][2])))
            plane = (chv + (ev >> 2)) * (H * W)
            sl = pl.ds(k * _L, _L)
            ia[sl] = plane + y0 * W + x0
            ib[sl] = plane + y0 * W + x1i
            ic[sl] = plane + y1i * W + x0
            id_[sl] = plane + y1i * W + x1i
        gather4(fd)
        for k in range(8):
            ly, lx = lws[k]
            sl = pl.ds(k * _L, _L)
            top = va[sl] * (1.0 - lx) + vb[sl] * lx
            bot = vc[sl] * (1.0 - lx) + vd[sl] * lx
            outv[sl] = top * (1.0 - ly) + bot * ly
        pltpu.sync_copy(outv, od.at[pl.ds((wid - t_lo) * 128, 128)])


def kernel(x_block1, x_block2, x_block3, x_block4, x_decoder, bbox):
    flat = [x.reshape(-1)
            for x in (x_block1, x_block2, x_block3, x_block4, x_decoder)]
    bb_b = jnp.broadcast_to(bbox.reshape(4, 1), (4, _L)).astype(jnp.float32)

    f32 = jnp.float32
    i32 = jnp.int32
    run = pl.kernel(
        _body,
        out_type=(
            jax.ShapeDtypeStruct((256,), f32),
            jax.ShapeDtypeStruct((512,), f32),
            jax.ShapeDtypeStruct((1024,), f32),
            jax.ShapeDtypeStruct((2048,), f32),
            jax.ShapeDtypeStruct((256,), f32),
        ),
        mesh=plsc.VectorSubcoreMesh(
            core_axis_name="c", subcore_axis_name="s",
            num_cores=_NC, num_subcores=_NS),
        scratch_types=[
            pltpu.VMEM((4, _L), f32),
            pltpu.VMEM((128,), i32), pltpu.VMEM((128,), i32),
            pltpu.VMEM((128,), i32), pltpu.VMEM((128,), i32),
            pltpu.VMEM((128,), f32), pltpu.VMEM((128,), f32),
            pltpu.VMEM((128,), f32), pltpu.VMEM((128,), f32),
            pltpu.VMEM((128,), f32),
            pltpu.SemaphoreType.DMA,
        ],
    )
    o1, o2, o3, o4, od = run(*flat, bb_b)
    return (
        o1.reshape(1, 256, 1, 1),
        o2.reshape(1, 512, 1, 1),
        o3.reshape(1, 1024, 1, 1),
        o4.reshape(1, 2048, 1, 1),
        od.reshape(1, 64, 2, 2),
    )


# speedup vs baseline: 4.3085x; 4.3085x over previous
import jax
import jax.numpy as jnp
from jax import lax
from jax.experimental import pallas as pl
from jax.experimental.pallas import tpu as pltpu
from jax.experimental.pallas import tpu_sc as plsc

_NC, _NS, _L = 2, 16, 16

_SC_LEVELS = (
    (64, 64, 8.0, 0, 4),
    (32, 32, 16.0, 4, 8),
    (16, 16, 32.0, 12, 16),
)


def _scal_split(pos, limit):
    pos = jnp.clip(pos, 0.0, float(limit - 1))
    i0 = pos.astype(jnp.int32)
    frac = pos - i0.astype(jnp.float32)
    i1 = jnp.minimum(i0 + 1, limit - 1)
    return i0, i1, frac


def _sc_body(f2, f3, f4, bbh, o2, o3, o4, bbv, va, vb, vc, vd, outv,
             s0, s1, s2, s3):
    wid = lax.axis_index("c") * _NS + lax.axis_index("s")
    pltpu.sync_copy(bbh, bbv)
    bx1 = bbv[0][0]
    by1 = bbv[1][0]
    bx2 = bbv[2][0]
    by2 = bbv[3][0]
    feats = (f2, f3, f4)
    outs = (o2, o3, o4)
    sems = (s0, s1, s2, s3)

    for lvl, (H, W, s, t_lo, n_t) in enumerate(_SC_LEVELS):
        @pl.when((wid >= t_lo) & (wid < t_lo + n_t))
        def _(lvl=lvl, H=H, W=W, s=s, t_lo=t_lo):
            feat, out = feats[lvl], outs[lvl]
            inv = 1.0 / s
            y0, y1i, ly = _scal_split((by1 + by2) * 0.5 * inv, H)
            x0, x1i, lx = _scal_split((bx1 + bx2) * 0.5 * inv, W)
            ch = pl.multiple_of((wid - t_lo) * 128, 128)
            srcs = (feat.at[0, y0, x0, pl.ds(ch, 128)],
                    feat.at[0, y0, x1i, pl.ds(ch, 128)],
                    feat.at[0, y1i, x0, pl.ds(ch, 128)],
                    feat.at[0, y1i, x1i, pl.ds(ch, 128)])
            cps = [pltpu.async_copy(s_, d_, sem)
                   for s_, d_, sem in zip(srcs, (va, vb, vc, vd), sems)]
            for cp in cps:
                cp.wait()
            lxv = jnp.full((_L,), lx, jnp.float32)
            lyv = jnp.full((_L,), ly, jnp.float32)
            for k in range(8):
                sl = pl.ds(k * _L, _L)
                top = va[sl] * (1.0 - lxv) + vb[sl] * lxv
                bot = vc[sl] * (1.0 - lxv) + vd[sl] * lxv
                outv[sl] = top * (1.0 - lyv) + bot * lyv
            pltpu.sync_copy(outv, out.at[pl.ds(ch, 128)])


_SLAB = 16


def _wplane(x0, x1i, lx, y0, y1i, ly, yb8, width):
    xl = lax.broadcasted_iota(jnp.int32, (_SLAB, width), 1)
    yl = lax.broadcasted_iota(jnp.int32, (_SLAB, width), 0) + yb8
    wx = jnp.where(xl == x0, 1.0 - lx, 0.0) + jnp.where(xl == x1i, lx, 0.0)
    wy = (jnp.where(yl == y0, 1.0 - ly, 0.0)
          + jnp.where(yl == y1i, ly, 0.0))
    return wx * wy


def _slab_base(y0, limit):
    return jnp.minimum((y0 // 8) * 8, limit - _SLAB)


def _tc_body(bb_ref, f1_ref, fd_ref, o1_ref, od_ref,
             slab1, slabd0, slabd1, sm1, smd0, smd1):
    bx1 = bb_ref[0, 0]
    by1 = bb_ref[0, 1]
    bx2 = bb_ref[0, 2]
    by2 = bb_ref[0, 3]

    y0, y1i, ly = _scal_split((by1 + by2) * 0.5 * 0.25, 128)
    x0, x1i, lx = _scal_split((bx1 + bx2) * 0.5 * 0.25, 128)
    yb = _slab_base(y0, 128)
    cp1 = pltpu.make_async_copy(
        f1_ref.at[0, :, pl.ds(yb, _SLAB), :], slab1, sm1)
    cp1.start()

    dy, dxs = [], []
    for i in range(2):
        yi0, yi1, lyi = _scal_split((by1 + (by2 - by1) * float(i)) * 0.5, 256)
        dy.append((yi0, yi1, lyi, _slab_base(yi0, 256)))
    for j in range(2):
        xj0, xj1, lxj = _scal_split((bx1 + (bx2 - bx1) * float(j)) * 0.5, 256)
        dxs.append((xj0, xj1, lxj))
    cpd0 = pltpu.make_async_copy(
        fd_ref.at[0, :, pl.ds(dy[0][3], _SLAB), :], slabd0, smd0)
    cpd0.start()
    cpd1 = pltpu.make_async_copy(
        fd_ref.at[0, :, pl.ds(dy[1][3], _SLAB), :], slabd1, smd1)
    cpd1.start()

    cp1.wait()
    w1 = _wplane(x0, x1i, lx, y0, y1i, ly, yb, 128)
    o1_ref[...] = jnp.sum(slab1[...] * w1[None, :, :], axis=(1, 2))

    cpd0.wait()
    cpd1.wait()
    for i, slab in ((0, slabd0), (1, slabd1)):
        yi0, yi1, lyi, ybi = dy[i]
        for j in range(2):
            xj0, xj1, lxj = dxs[j]
            wij = _wplane(xj0, xj1, lxj, yi0, yi1, lyi, ybi, 256)
            od_ref[i, j] = jnp.sum(slab[...] * wij[None, :, :], axis=(1, 2))


def kernel(x_block1, x_block2, x_block3, x_block4, x_decoder, bbox):
    f2t = jnp.transpose(x_block2, (0, 2, 3, 1))
    f3t = jnp.transpose(x_block3, (0, 2, 3, 1))
    f4t = jnp.transpose(x_block4, (0, 2, 3, 1))
    bb_b = jnp.broadcast_to(bbox.reshape(4, 1), (4, _L)).astype(jnp.float32)

    f32 = jnp.float32
    sc_run = pl.kernel(
        _sc_body,
        out_type=(
            jax.ShapeDtypeStruct((512,), f32),
            jax.ShapeDtypeStruct((1024,), f32),
            jax.ShapeDtypeStruct((2048,), f32),
        ),
        mesh=plsc.VectorSubcoreMesh(
            core_axis_name="c", subcore_axis_name="s",
            num_cores=_NC, num_subcores=_NS),
        scratch_types=[
            pltpu.VMEM((4, _L), f32),
            pltpu.VMEM((128,), f32), pltpu.VMEM((128,), f32),
            pltpu.VMEM((128,), f32), pltpu.VMEM((128,), f32),
            pltpu.VMEM((128,), f32),
            pltpu.SemaphoreType.DMA, pltpu.SemaphoreType.DMA,
            pltpu.SemaphoreType.DMA, pltpu.SemaphoreType.DMA,
        ],
    )
    o2, o3, o4 = sc_run(f2t, f3t, f4t, bb_b)

    tc_run = pl.pallas_call(
        _tc_body,
        out_shape=(
            jax.ShapeDtypeStruct((256,), f32),
            jax.ShapeDtypeStruct((2, 2, 64), f32),
        ),
        in_specs=[
            pl.BlockSpec(memory_space=pltpu.SMEM),
            pl.BlockSpec(memory_space=pltpu.MemorySpace.HBM),
            pl.BlockSpec(memory_space=pltpu.MemorySpace.HBM),
        ],
        out_specs=(
            pl.BlockSpec(memory_space=pltpu.VMEM),
            pl.BlockSpec(memory_space=pltpu.VMEM),
        ),
        scratch_shapes=[
            pltpu.VMEM((256, 16, 128), f32),
            pltpu.VMEM((64, 16, 256), f32),
            pltpu.VMEM((64, 16, 256), f32),
            pltpu.SemaphoreType.DMA,
            pltpu.SemaphoreType.DMA,
            pltpu.SemaphoreType.DMA,
        ],
    )
    o1, od22 = tc_run(bbox, x_block1, x_decoder)
    od = jnp.transpose(od22, (2, 0, 1))
    return (
        o1.reshape(1, 256, 1, 1),
        o2.reshape(1, 512, 1, 1),
        o3.reshape(1, 1024, 1, 1),
        o4.reshape(1, 2048, 1, 1),
        od.reshape(1, 64, 2, 2),
    )


# speedup vs baseline: 4.4857x; 1.0411x over previous
import jax
import jax.numpy as jnp
from jax import lax
from jax.experimental import pallas as pl
from jax.experimental.pallas import tpu as pltpu
from jax.experimental.pallas import tpu_sc as plsc

_NC, _NS, _L = 2, 16, 16

_SC_LEVELS = (
    (64, 64, 8.0, 0, 4),
    (32, 32, 16.0, 4, 8),
    (16, 16, 32.0, 12, 16),
)


def _scal_split(pos, limit):
    hi = limit - 1
    pos = jnp.clip(pos, 0.0, hi * 1.0)
    i0 = pos.astype(jnp.int32)
    frac = pos - i0.astype(jnp.float32)
    i1 = jnp.minimum(i0 + 1, hi)
    return i0, i1, frac


def _sc_body(f2, f3, f4, bbh, o2, o3, o4, bbv, va, vb, vc, vd, outv,
             s0, s1, s2, s3):
    wid = lax.axis_index("c") * _NS + lax.axis_index("s")
    pltpu.sync_copy(bbh.at[0], bbv.at[pl.ds(0, 4)])
    bb = bbv[...]
    bx1 = bb[0]
    by1 = bb[1]
    bx2 = bb[2]
    by2 = bb[3]
    sems = (s0, s1, s2, s3)

    def sel(vals):
        v2, v3, v4 = vals
        return jnp.where(wid < 4, v2, jnp.where(wid < 12, v3, v4))

    H = sel([l[0] for l in _SC_LEVELS])
    W = sel([l[1] for l in _SC_LEVELS])
    inv = sel([1.0 / l[2] for l in _SC_LEVELS])
    t_lo = sel([l[3] for l in _SC_LEVELS])
    y0, y1i, ly = _scal_split((by1 + by2) * 0.5 * inv, H)
    x0, x1i, lx = _scal_split((bx1 + bx2) * 0.5 * inv, W)
    ch = pl.multiple_of((wid - t_lo) * 128, 128)

    for lvl, feat in enumerate((f2, f3, f4)):
        _, _, _, lo, n_t = _SC_LEVELS[lvl]

        @pl.when((wid >= lo) & (wid < lo + n_t))
        def _(feat=feat):
            srcs = (feat.at[0, y0, x0, pl.ds(ch, 128)],
                    feat.at[0, y0, x1i, pl.ds(ch, 128)],
                    feat.at[0, y1i, x0, pl.ds(ch, 128)],
                    feat.at[0, y1i, x1i, pl.ds(ch, 128)])
            for s_, d_, sem in zip(srcs, (va, vb, vc, vd), sems):
                pltpu.async_copy(s_, d_, sem)

    @pl.when(wid < 28)
    def _():
        for sem, d_ in zip(sems, (va, vb, vc, vd)):
            pltpu.make_async_copy(f2.at[0, 0, 0, pl.ds(0, 128)], d_,
                                  sem).wait()
        lxv = jnp.full((_L,), lx, jnp.float32)
        lyv = jnp.full((_L,), ly, jnp.float32)
        for k in range(8):
            sl = pl.ds(k * _L, _L)
            top = va[sl] * (1.0 - lxv) + vb[sl] * lxv
            bot = vc[sl] * (1.0 - lxv) + vd[sl] * lxv
            outv[sl] = top * (1.0 - lyv) + bot * lyv

    for lvl, out in enumerate((o2, o3, o4)):
        _, _, _, lo, n_t = _SC_LEVELS[lvl]

        @pl.when((wid >= lo) & (wid < lo + n_t))
        def _(out=out):
            pltpu.sync_copy(outv, out.at[pl.ds(ch, 128)])


_SLAB = 16


def _wplane(x0, x1i, lx, y0, y1i, ly, yb8, width):
    xl = lax.broadcasted_iota(jnp.int32, (_SLAB, width), 1)
    yl = lax.broadcasted_iota(jnp.int32, (_SLAB, width), 0) + yb8
    wx = jnp.where(xl == x0, 1.0 - lx, 0.0) + jnp.where(xl == x1i, lx, 0.0)
    wy = (jnp.where(yl == y0, 1.0 - ly, 0.0)
          + jnp.where(yl == y1i, ly, 0.0))
    return wx * wy


def _slab_base(y0, limit):
    return jnp.minimum((y0 // 8) * 8, limit - _SLAB)


def _tc_body(bb_ref, f1_ref, fd_ref, o1_ref, od_ref,
             slab1, slabd0, slabd1, sm1, smd0, smd1):
    bx1 = bb_ref[0, 0]
    by1 = bb_ref[0, 1]
    bx2 = bb_ref[0, 2]
    by2 = bb_ref[0, 3]

    y0, y1i, ly = _scal_split((by1 + by2) * 0.5 * 0.25, 128)
    x0, x1i, lx = _scal_split((bx1 + bx2) * 0.5 * 0.25, 128)
    yb = _slab_base(y0, 128)
    cp1 = pltpu.make_async_copy(
        f1_ref.at[0, :, pl.ds(yb, _SLAB), :], slab1, sm1)
    cp1.start()

    dy, dxs = [], []
    for i in range(2):
        yi0, yi1, lyi = _scal_split((by1 + (by2 - by1) * float(i)) * 0.5, 256)
        dy.append((yi0, yi1, lyi, _slab_base(yi0, 256)))
    for j in range(2):
        xj0, xj1, lxj = _scal_split((bx1 + (bx2 - bx1) * float(j)) * 0.5, 256)
        dxs.append((xj0, xj1, lxj))
    cpd0 = pltpu.make_async_copy(
        fd_ref.at[0, :, pl.ds(dy[0][3], _SLAB), :], slabd0, smd0)
    cpd0.start()
    cpd1 = pltpu.make_async_copy(
        fd_ref.at[0, :, pl.ds(dy[1][3], _SLAB), :], slabd1, smd1)
    cpd1.start()

    cp1.wait()
    w1 = _wplane(x0, x1i, lx, y0, y1i, ly, yb, 128)
    o1_ref[...] = jnp.sum(slab1[...] * w1[None, :, :], axis=(1, 2))

    cpd0.wait()
    cpd1.wait()
    for i, slab in ((0, slabd0), (1, slabd1)):
        yi0, yi1, lyi, ybi = dy[i]
        for j in range(2):
            xj0, xj1, lxj = dxs[j]
            wij = _wplane(xj0, xj1, lxj, yi0, yi1, lyi, ybi, 256)
            od_ref[i, j] = jnp.sum(slab[...] * wij[None, :, :], axis=(1, 2))


def kernel(x_block1, x_block2, x_block3, x_block4, x_decoder, bbox):
    f2t = jnp.transpose(x_block2, (0, 2, 3, 1))
    f3t = jnp.transpose(x_block3, (0, 2, 3, 1))
    f4t = jnp.transpose(x_block4, (0, 2, 3, 1))

    f32 = jnp.float32
    sc_run = pl.kernel(
        _sc_body,
        out_type=(
            jax.ShapeDtypeStruct((512,), f32),
            jax.ShapeDtypeStruct((1024,), f32),
            jax.ShapeDtypeStruct((2048,), f32),
        ),
        mesh=plsc.VectorSubcoreMesh(
            core_axis_name="c", subcore_axis_name="s",
            num_cores=_NC, num_subcores=_NS),
        scratch_types=[
            pltpu.VMEM((_L,), f32),
            pltpu.VMEM((128,), f32), pltpu.VMEM((128,), f32),
            pltpu.VMEM((128,), f32), pltpu.VMEM((128,), f32),
            pltpu.VMEM((128,), f32),
            pltpu.SemaphoreType.DMA, pltpu.SemaphoreType.DMA,
            pltpu.SemaphoreType.DMA, pltpu.SemaphoreType.DMA,
        ],
    )
    o2, o3, o4 = sc_run(f2t, f3t, f4t, bbox)

    tc_run = pl.pallas_call(
        _tc_body,
        out_shape=(
            jax.ShapeDtypeStruct((256,), f32),
            jax.ShapeDtypeStruct((2, 2, 64), f32),
        ),
        in_specs=[
            pl.BlockSpec(memory_space=pltpu.SMEM),
            pl.BlockSpec(memory_space=pltpu.MemorySpace.HBM),
            pl.BlockSpec(memory_space=pltpu.MemorySpace.HBM),
        ],
        out_specs=(
            pl.BlockSpec(memory_space=pltpu.VMEM),
            pl.BlockSpec(memory_space=pltpu.VMEM),
        ),
        scratch_shapes=[
            pltpu.VMEM((256, 16, 128), f32),
            pltpu.VMEM((64, 16, 256), f32),
            pltpu.VMEM((64, 16, 256), f32),
            pltpu.SemaphoreType.DMA,
            pltpu.SemaphoreType.DMA,
            pltpu.SemaphoreType.DMA,
        ],
    )
    o1, od22 = tc_run(bbox, x_block1, x_decoder)
    od = jnp.transpose(od22, (2, 0, 1))
    return (
        o1.reshape(1, 256, 1, 1),
        o2.reshape(1, 512, 1, 1),
        o3.reshape(1, 1024, 1, 1),
        o4.reshape(1, 2048, 1, 1),
        od.reshape(1, 64, 2, 2),
    )


# speedup vs baseline: 4.7666x; 1.0626x over previous
import jax
import jax.numpy as jnp
from jax import lax
from jax.experimental import pallas as pl
from jax.experimental.pallas import tpu as pltpu
from jax.experimental.pallas import tpu_sc as plsc

_NC, _NS, _L = 2, 16, 16

_SC_LEVELS = (
    (64, 64, 8.0, 0, 2),
    (32, 32, 16.0, 2, 4),
    (16, 16, 32.0, 6, 8),
)


def _scal_split(pos, limit):
    hi = limit - 1
    pos = jnp.clip(pos, 0.0, hi * 1.0)
    i0 = pos.astype(jnp.int32)
    frac = pos - i0.astype(jnp.float32)
    i1 = jnp.minimum(i0 + 1, hi)
    return i0, i1, frac


def _sc_body(f2, f3, f4, bbh, o2, o3, o4, bbv, va, vb, vc, vd, outv,
             s0, s1, s2, s3):
    wid = lax.axis_index("c") * _NS + lax.axis_index("s")
    pltpu.sync_copy(bbh.at[0], bbv.at[pl.ds(0, 4)])
    bb = bbv[...]
    bx1 = bb[0]
    by1 = bb[1]
    bx2 = bb[2]
    by2 = bb[3]
    sems = (s0, s1, s2, s3)

    def sel(vals):
        v2, v3, v4 = vals
        return jnp.where(wid < 2, v2, jnp.where(wid < 6, v3, v4))

    H = sel([l[0] for l in _SC_LEVELS])
    W = sel([l[1] for l in _SC_LEVELS])
    inv = sel([1.0 / l[2] for l in _SC_LEVELS])
    t_lo = sel([l[3] for l in _SC_LEVELS])
    y0, y1i, ly = _scal_split((by1 + by2) * 0.5 * inv, H)
    x0, x1i, lx = _scal_split((bx1 + bx2) * 0.5 * inv, W)
    ch = pl.multiple_of((wid - t_lo) * 256, 256)

    for lvl, feat in enumerate((f2, f3, f4)):
        _, _, _, lo, n_t = _SC_LEVELS[lvl]

        @pl.when((wid >= lo) & (wid < lo + n_t))
        def _(feat=feat):
            srcs = (feat.at[0, y0, x0, pl.ds(ch, 256)],
                    feat.at[0, y0, x1i, pl.ds(ch, 256)],
                    feat.at[0, y1i, x0, pl.ds(ch, 256)],
                    feat.at[0, y1i, x1i, pl.ds(ch, 256)])
            for s_, d_, sem in zip(srcs, (va, vb, vc, vd), sems):
                pltpu.async_copy(s_, d_, sem)

    @pl.when(wid < 14)
    def _():
        for sem, d_ in zip(sems, (va, vb, vc, vd)):
            pltpu.make_async_copy(f2.at[0, 0, 0, pl.ds(0, 256)], d_,
                                  sem).wait()
        lxv = jnp.full((_L,), lx, jnp.float32)
        lyv = jnp.full((_L,), ly, jnp.float32)
        for k in range(16):
            sl = pl.ds(k * _L, _L)
            top = va[sl] * (1.0 - lxv) + vb[sl] * lxv
            bot = vc[sl] * (1.0 - lxv) + vd[sl] * lxv
            outv[sl] = top * (1.0 - lyv) + bot * lyv

    for lvl, out in enumerate((o2, o3, o4)):
        _, _, _, lo, n_t = _SC_LEVELS[lvl]

        @pl.when((wid >= lo) & (wid < lo + n_t))
        def _(out=out):
            pltpu.sync_copy(outv, out.at[pl.ds(ch, 256)])


_SLAB = 16


def _wplane(x0, x1i, lx, y0, y1i, ly, yb8, width):
    xl = lax.broadcasted_iota(jnp.int32, (_SLAB, width), 1)
    yl = lax.broadcasted_iota(jnp.int32, (_SLAB, width), 0) + yb8
    wx = jnp.where(xl == x0, 1.0 - lx, 0.0) + jnp.where(xl == x1i, lx, 0.0)
    wy = (jnp.where(yl == y0, 1.0 - ly, 0.0)
          + jnp.where(yl == y1i, ly, 0.0))
    return wx * wy


def _slab_base(y0, limit):
    return jnp.minimum((y0 // 8) * 8, limit - _SLAB)


def _tc_body(bb_ref, f1_ref, fd_ref, o1_ref, od_ref,
             slab1, slabd0, slabd1, sm1, smd0, smd1):
    bx1 = bb_ref[0, 0]
    by1 = bb_ref[0, 1]
    bx2 = bb_ref[0, 2]
    by2 = bb_ref[0, 3]

    y0, y1i, ly = _scal_split((by1 + by2) * 0.5 * 0.25, 128)
    x0, x1i, lx = _scal_split((bx1 + bx2) * 0.5 * 0.25, 128)
    yb = _slab_base(y0, 128)
    cp1 = pltpu.make_async_copy(
        f1_ref.at[0, :, pl.ds(yb, _SLAB), :], slab1, sm1)
    cp1.start()

    dy, dxs = [], []
    for i in range(2):
        yi0, yi1, lyi = _scal_split((by1 + (by2 - by1) * float(i)) * 0.5, 256)
        dy.append((yi0, yi1, lyi, _slab_base(yi0, 256)))
    for j in range(2):
        xj0, xj1, lxj = _scal_split((bx1 + (bx2 - bx1) * float(j)) * 0.5, 256)
        dxs.append((xj0, xj1, lxj))
    cpd0 = pltpu.make_async_copy(
        fd_ref.at[0, :, pl.ds(dy[0][3], _SLAB), :], slabd0, smd0)
    cpd0.start()
    cpd1 = pltpu.make_async_copy(
        fd_ref.at[0, :, pl.ds(dy[1][3], _SLAB), :], slabd1, smd1)
    cpd1.start()

    cp1.wait()
    w1 = _wplane(x0, x1i, lx, y0, y1i, ly, yb, 128)
    o1_ref[...] = jnp.sum(slab1[...] * w1[None, :, :], axis=(1, 2))

    cpd0.wait()
    cpd1.wait()
    for i, slab in ((0, slabd0), (1, slabd1)):
        yi0, yi1, lyi, ybi = dy[i]
        for j in range(2):
            xj0, xj1, lxj = dxs[j]
            wij = _wplane(xj0, xj1, lxj, yi0, yi1, lyi, ybi, 256)
            od_ref[i, j] = jnp.sum(slab[...] * wij[None, :, :], axis=(1, 2))


def kernel(x_block1, x_block2, x_block3, x_block4, x_decoder, bbox):
    f2t = jnp.transpose(x_block2, (0, 2, 3, 1))
    f3t = jnp.transpose(x_block3, (0, 2, 3, 1))
    f4t = jnp.transpose(x_block4, (0, 2, 3, 1))

    f32 = jnp.float32
    sc_run = pl.kernel(
        _sc_body,
        out_type=(
            jax.ShapeDtypeStruct((512,), f32),
            jax.ShapeDtypeStruct((1024,), f32),
            jax.ShapeDtypeStruct((2048,), f32),
        ),
        mesh=plsc.VectorSubcoreMesh(
            core_axis_name="c", subcore_axis_name="s",
            num_cores=1, num_subcores=_NS),
        scratch_types=[
            pltpu.VMEM((_L,), f32),
            pltpu.VMEM((256,), f32), pltpu.VMEM((256,), f32),
            pltpu.VMEM((256,), f32), pltpu.VMEM((256,), f32),
            pltpu.VMEM((256,), f32),
            pltpu.SemaphoreType.DMA, pltpu.SemaphoreType.DMA,
            pltpu.SemaphoreType.DMA, pltpu.SemaphoreType.DMA,
        ],
    )
    o2, o3, o4 = sc_run(f2t, f3t, f4t, bbox)

    tc_run = pl.pallas_call(
        _tc_body,
        out_shape=(
            jax.ShapeDtypeStruct((256,), f32),
            jax.ShapeDtypeStruct((2, 2, 64), f32),
        ),
        in_specs=[
            pl.BlockSpec(memory_space=pltpu.SMEM),
            pl.BlockSpec(memory_space=pltpu.MemorySpace.HBM),
            pl.BlockSpec(memory_space=pltpu.MemorySpace.HBM),
        ],
        out_specs=(
            pl.BlockSpec(memory_space=pltpu.VMEM),
            pl.BlockSpec(memory_space=pltpu.VMEM),
        ),
        scratch_shapes=[
            pltpu.VMEM((256, 16, 128), f32),
            pltpu.VMEM((64, 16, 256), f32),
            pltpu.VMEM((64, 16, 256), f32),
            pltpu.SemaphoreType.DMA,
            pltpu.SemaphoreType.DMA,
            pltpu.SemaphoreType.DMA,
        ],
    )
    o1, od22 = tc_run(bbox, x_block1, x_decoder)
    od = jnp.transpose(od22, (2, 0, 1))
    return (
        o1.reshape(1, 256, 1, 1),
        o2.reshape(1, 512, 1, 1),
        o3.reshape(1, 1024, 1, 1),
        o4.reshape(1, 2048, 1, 1),
        od.reshape(1, 64, 2, 2),
    )


# speedup vs baseline: 4.7752x; 1.0018x over previous
import jax
import jax.numpy as jnp
from jax import lax
from jax.experimental import pallas as pl
from jax.experimental.pallas import tpu as pltpu
from jax.experimental.pallas import tpu_sc as plsc

_NC, _NS, _L = 2, 16, 16

_SC_LEVELS = (
    (64, 64, 8.0, 0, 2),
    (32, 32, 16.0, 2, 4),
    (16, 16, 32.0, 6, 8),
)


def _scal_split(pos, limit):
    hi = limit - 1
    pos = jnp.clip(pos, 0.0, hi * 1.0)
    i0 = pos.astype(jnp.int32)
    frac = pos - i0.astype(jnp.float32)
    i1 = jnp.minimum(i0 + 1, hi)
    return i0, i1, frac


def _sc_body(f2, f3, f4, bbh, o2, o3, o4, bbv, va, vb, vc, vd, outv,
             s0, s1, s2, s3):
    wid = lax.axis_index("c") * _NS + lax.axis_index("s")
    pltpu.sync_copy(bbh.at[0], bbv.at[pl.ds(0, 4)])
    bb = bbv[...]
    bx1 = bb[0]
    by1 = bb[1]
    bx2 = bb[2]
    by2 = bb[3]
    sems = (s0, s1, s2, s3)

    def sel(vals):
        v2, v3, v4 = vals
        return jnp.where(wid < 2, v2, jnp.where(wid < 6, v3, v4))

    H = sel([l[0] for l in _SC_LEVELS])
    W = sel([l[1] for l in _SC_LEVELS])
    inv = sel([1.0 / l[2] for l in _SC_LEVELS])
    t_lo = sel([l[3] for l in _SC_LEVELS])
    y0, y1i, ly = _scal_split((by1 + by2) * 0.5 * inv, H)
    x0, x1i, lx = _scal_split((bx1 + bx2) * 0.5 * inv, W)
    ch = pl.multiple_of((wid - t_lo) * 256, 256)

    for lvl, feat in enumerate((f2, f3, f4)):
        _, _, _, lo, n_t = _SC_LEVELS[lvl]

        @pl.when((wid >= lo) & (wid < lo + n_t))
        def _(feat=feat):
            srcs = (feat.at[0, y0, x0, pl.ds(ch, 256)],
                    feat.at[0, y0, x1i, pl.ds(ch, 256)],
                    feat.at[0, y1i, x0, pl.ds(ch, 256)],
                    feat.at[0, y1i, x1i, pl.ds(ch, 256)])
            for s_, d_, sem in zip(srcs, (va, vb, vc, vd), sems):
                pltpu.async_copy(s_, d_, sem)

    @pl.when(wid < 14)
    def _():
        for sem, d_ in zip(sems, (va, vb, vc, vd)):
            pltpu.make_async_copy(f2.at[0, 0, 0, pl.ds(0, 256)], d_,
                                  sem).wait()
        lxv = jnp.full((_L,), lx, jnp.float32)
        lyv = jnp.full((_L,), ly, jnp.float32)

        @pl.loop(0, 256, step=_L)
        def _(k):
            sl = pl.ds(k, _L)
            top = va[sl] * (1.0 - lxv) + vb[sl] * lxv
            bot = vc[sl] * (1.0 - lxv) + vd[sl] * lxv
            outv[sl] = top * (1.0 - lyv) + bot * lyv

    for lvl, out in enumerate((o2, o3, o4)):
        _, _, _, lo, n_t = _SC_LEVELS[lvl]

        @pl.when((wid >= lo) & (wid < lo + n_t))
        def _(out=out):
            pltpu.sync_copy(outv, out.at[pl.ds(ch, 256)])


_SLAB = 16


def _wplane(x0, x1i, lx, y0, y1i, ly, yb8, width):
    xl = lax.broadcasted_iota(jnp.int32, (_SLAB, width), 1)
    yl = lax.broadcasted_iota(jnp.int32, (_SLAB, width), 0) + yb8
    wx = jnp.where(xl == x0, 1.0 - lx, 0.0) + jnp.where(xl == x1i, lx, 0.0)
    wy = (jnp.where(yl == y0, 1.0 - ly, 0.0)
          + jnp.where(yl == y1i, ly, 0.0))
    return wx * wy


def _slab_base(y0, limit):
    return jnp.minimum((y0 // 8) * 8, limit - _SLAB)


def _tc_body(bb_ref, f1_ref, fd_ref, o1_ref, od_ref,
             slab1, slabd0, slabd1, sm1, smd0, smd1):
    bx1 = bb_ref[0, 0]
    by1 = bb_ref[0, 1]
    bx2 = bb_ref[0, 2]
    by2 = bb_ref[0, 3]

    y0, y1i, ly = _scal_split((by1 + by2) * 0.5 * 0.25, 128)
    x0, x1i, lx = _scal_split((bx1 + bx2) * 0.5 * 0.25, 128)
    yb = _slab_base(y0, 128)
    cp1 = pltpu.make_async_copy(
        f1_ref.at[0, :, pl.ds(yb, _SLAB), :], slab1, sm1)
    cp1.start()

    dy, dxs = [], []
    for i in range(2):
        yi0, yi1, lyi = _scal_split((by1 + (by2 - by1) * float(i)) * 0.5, 256)
        dy.append((yi0, yi1, lyi, _slab_base(yi0, 256)))
    for j in range(2):
        xj0, xj1, lxj = _scal_split((bx1 + (bx2 - bx1) * float(j)) * 0.5, 256)
        dxs.append((xj0, xj1, lxj))
    cpd0 = pltpu.make_async_copy(
        fd_ref.at[0, :, pl.ds(dy[0][3], _SLAB), :], slabd0, smd0)
    cpd0.start()
    cpd1 = pltpu.make_async_copy(
        fd_ref.at[0, :, pl.ds(dy[1][3], _SLAB), :], slabd1, smd1)
    cpd1.start()

    cp1.wait()
    w1 = _wplane(x0, x1i, lx, y0, y1i, ly, yb, 128)
    o1_ref[...] = jnp.sum(slab1[...] * w1[None, :, :], axis=(1, 2))

    cpd0.wait()
    cpd1.wait()
    for i, slab in ((0, slabd0), (1, slabd1)):
        yi0, yi1, lyi, ybi = dy[i]
        for j in range(2):
            xj0, xj1, lxj = dxs[j]
            wij = _wplane(xj0, xj1, lxj, yi0, yi1, lyi, ybi, 256)
            od_ref[i, j] = jnp.sum(slab[...] * wij[None, :, :], axis=(1, 2))


def kernel(x_block1, x_block2, x_block3, x_block4, x_decoder, bbox):
    f2t = jnp.transpose(x_block2, (0, 2, 3, 1))
    f3t = jnp.transpose(x_block3, (0, 2, 3, 1))
    f4t = jnp.transpose(x_block4, (0, 2, 3, 1))

    f32 = jnp.float32
    sc_run = pl.kernel(
        _sc_body,
        out_type=(
            jax.ShapeDtypeStruct((512,), f32),
            jax.ShapeDtypeStruct((1024,), f32),
            jax.ShapeDtypeStruct((2048,), f32),
        ),
        mesh=plsc.VectorSubcoreMesh(
            core_axis_name="c", subcore_axis_name="s",
            num_cores=1, num_subcores=_NS),
        scratch_types=[
            pltpu.VMEM((_L,), f32),
            pltpu.VMEM((256,), f32), pltpu.VMEM((256,), f32),
            pltpu.VMEM((256,), f32), pltpu.VMEM((256,), f32),
            pltpu.VMEM((256,), f32),
            pltpu.SemaphoreType.DMA, pltpu.SemaphoreType.DMA,
            pltpu.SemaphoreType.DMA, pltpu.SemaphoreType.DMA,
        ],
    )
    o2, o3, o4 = sc_run(f2t, f3t, f4t, bbox)

    tc_run = pl.pallas_call(
        _tc_body,
        out_shape=(
            jax.ShapeDtypeStruct((256,), f32),
            jax.ShapeDtypeStruct((2, 2, 64), f32),
        ),
        in_specs=[
            pl.BlockSpec(memory_space=pltpu.SMEM),
            pl.BlockSpec(memory_space=pltpu.MemorySpace.HBM),
            pl.BlockSpec(memory_space=pltpu.MemorySpace.HBM),
        ],
        out_specs=(
            pl.BlockSpec(memory_space=pltpu.VMEM),
            pl.BlockSpec(memory_space=pltpu.VMEM),
        ),
        scratch_shapes=[
            pltpu.VMEM((256, 16, 128), f32),
            pltpu.VMEM((64, 16, 256), f32),
            pltpu.VMEM((64, 16, 256), f32),
            pltpu.SemaphoreType.DMA,
            pltpu.SemaphoreType.DMA,
            pltpu.SemaphoreType.DMA,
        ],
    )
    o1, od22 = tc_run(bbox, x_block1, x_decoder)
    od = jnp.transpose(od22, (2, 0, 1))
    return (
        o1.reshape(1, 256, 1, 1),
        o2.reshape(1, 512, 1, 1),
        o3.reshape(1, 1024, 1, 1),
        o4.reshape(1, 2048, 1, 1),
        od.reshape(1, 64, 2, 2),
    )
